# baseline (speedup 1.0000x reference)
import functools

import jax
import jax.numpy as jnp
from jax.experimental import pallas as pl
from jax.experimental.pallas import tpu as pltpu

B, D, T = 16, 256, 1024
NQ, K = 8, 1024
TT = 256
NT = T // TT


def _row_sum_sq(v):
    rr = v * v
    pre = rr[:, 0:128] + rr[:, 128:256]
    acc = pre[:, 0:8]
    for a in range(1, 16):
        acc = acc + pre[:, 8 * a:8 * a + 8]
    s = acc[:, 0:4] + acc[:, 4:8]
    s = s[:, 0:2] + s[:, 2:4]
    return s[:, 0:1] + s[:, 1:2]


def _rvq_kernel(z_ref, cb_ref, cn2_ref, layers_ref, zq_ref, emb_ref,
                counts_ref, commit_ref, cbh_ref, cbm_ref, cbl_ref):
    first = (pl.program_id(0) == 0) & (pl.program_id(1) == 0)

    @pl.when(first)
    def _init():
        counts_ref[...] = jnp.zeros_like(counts_ref)
        commit_ref[...] = jnp.zeros_like(commit_ref)
        cb = cb_ref[...]
        hi = cb.astype(jnp.bfloat16)
        rem = cb - hi.astype(jnp.float32)
        mid = rem.astype(jnp.bfloat16)
        lo = (rem - mid.astype(jnp.float32)).astype(jnp.bfloat16)
        cbh_ref[...] = hi
        cbm_ref[...] = mid
        cbl_ref[...] = lo

    x = z_ref[0]
    r = x.T
    qo = jnp.zeros((TT, D), jnp.float32)
    iota_f = jax.lax.broadcasted_iota(jnp.int32, (TT, K), 1).astype(jnp.float32)
    dims = (((1,), (0,)), ((), ()))
    for i in range(NQ):
        cb = cb_ref[i]
        mm = jax.lax.dot_general(
            r, cb, (((1,), (1,)), ((), ())),
            precision=jax.lax.Precision.DEFAULT)
        rn2 = _row_sum_sq(r)
        dist = (rn2 - 2.0 * mm) + cn2_ref[i][None, :]
        minv = jnp.min(dist, axis=1, keepdims=True)
        idxf = jnp.min(jnp.where(dist == minv, iota_f, jnp.float32(K)),
                       axis=1, keepdims=True)
        oh = (iota_f == idxf).astype(jnp.float32)
        ohb = oh.astype(jnp.bfloat16)
        q = ((jax.lax.dot_general(ohb, cbh_ref[i], dims,
                                  preferred_element_type=jnp.float32)
              + jax.lax.dot_general(ohb, cbm_ref[i], dims,
                                    preferred_element_type=jnp.float32))
             + jax.lax.dot_general(ohb, cbl_ref[i], dims,
                                   preferred_element_type=jnp.float32))
        d = q - r
        qo = qo + (r + d)
        commit_ref[i, :] += jnp.sum(d * d, axis=0)
        counts_ref[i, :] += jnp.sum(oh, axis=0)
        r = r - q
        emb_ref[0, i, :] = idxf[:, 0].astype(jnp.int32)
        layers_ref[i, 0, :, :] = qo.T
    zq_ref[0] = layers_ref[NQ - 1, 0, :, :]


@functools.partial(jax.jit, static_argnames=())
def kernel(z, codebooks):
    cn2 = jnp.stack([jnp.sum(codebooks[i] ** 2, axis=-1)
                     for i in range(NQ)])
    layers, zq, emb, counts, commit = pl.pallas_call(
        _rvq_kernel,
        grid=(B, NT),
        in_specs=[
            pl.BlockSpec((1, D, TT), lambda b, t: (b, 0, t)),
            pl.BlockSpec((NQ, K, D), lambda b, t: (0, 0, 0)),
            pl.BlockSpec((NQ, K), lambda b, t: (0, 0)),
        ],
        out_specs=[
            pl.BlockSpec((NQ, 1, D, TT), lambda b, t: (0, b, 0, t)),
            pl.BlockSpec((1, D, TT), lambda b, t: (b, 0, t)),
            pl.BlockSpec((1, NQ, TT), lambda b, t: (b, 0, t)),
            pl.BlockSpec((NQ, K), lambda b, t: (0, 0)),
            pl.BlockSpec((NQ, D), lambda b, t: (0, 0)),
        ],
        out_shape=[
            jax.ShapeDtypeStruct((NQ, B, D, T), jnp.float32),
            jax.ShapeDtypeStruct((B, D, T), jnp.float32),
            jax.ShapeDtypeStruct((B, NQ, T), jnp.int32),
            jax.ShapeDtypeStruct((NQ, K), jnp.float32),
            jax.ShapeDtypeStruct((NQ, D), jnp.float32),
        ],
        scratch_shapes=[pltpu.VMEM((NQ, K, D), jnp.bfloat16),
                        pltpu.VMEM((NQ, K, D), jnp.bfloat16),
                        pltpu.VMEM((NQ, K, D), jnp.bfloat16)],
        compiler_params=pltpu.CompilerParams(
            dimension_semantics=("arbitrary", "arbitrary")),
    )(z, codebooks, cn2)

    embed_nums = jnp.transpose(emb, (1, 0, 2))
    vqloss = jnp.sum(commit) / jnp.float32(B * T * D)
    probs = counts / jnp.float32(B * T)
    perps = jnp.exp(-jnp.sum(probs * jnp.log(probs + 1e-10), axis=1))
    perplexity = jnp.mean(perps)
    return (zq, embed_nums, vqloss, perplexity, layers)


# speedup vs baseline: 1.3015x; 1.3015x over previous
import functools

import jax
import jax.numpy as jnp
from jax.experimental import pallas as pl
from jax.experimental.pallas import tpu as pltpu

B, D, T = 16, 256, 1024
NQ, K = 8, 1024
TT = 256
NT = T // TT


def _row_sum_sq(v):
    vt = v.T
    rr = vt * vt
    pre = rr[0:128, :] + rr[128:256, :]
    acc = pre[0:8, :]
    for a in range(1, 16):
        acc = acc + pre[8 * a:8 * a + 8, :]
    s = acc[0:4, :] + acc[4:8, :]
    s = s[0:2, :] + s[2:4, :]
    return (s[0:1, :] + s[1:2, :]).T


def _rvq_kernel(z_ref, cb_ref, cn2_ref, layers_ref, zq_ref, emb_ref,
                counts_ref, commit_ref, cbh_ref, cbm_ref, cbl_ref):
    first = (pl.program_id(0) == 0) & (pl.program_id(1) == 0)

    @pl.when(first)
    def _init():
        counts_ref[...] = jnp.zeros_like(counts_ref)
        commit_ref[...] = jnp.zeros_like(commit_ref)
        cb = cb_ref[...]
        hi = cb.astype(jnp.bfloat16)
        rem = cb - hi.astype(jnp.float32)
        mid = rem.astype(jnp.bfloat16)
        lo = (rem - mid.astype(jnp.float32)).astype(jnp.bfloat16)
        cbh_ref[...] = hi
        cbm_ref[...] = mid
        cbl_ref[...] = lo

    x = z_ref[0]
    r = x.T
    qo = jnp.zeros((TT, D), jnp.float32)
    iota_f = jax.lax.broadcasted_iota(jnp.int32, (TT, K), 1).astype(jnp.float32)
    dims = (((1,), (0,)), ((), ()))
    for i in range(NQ):
        cb = cb_ref[i]
        mm = jax.lax.dot_general(
            r, cb, (((1,), (1,)), ((), ())),
            precision=jax.lax.Precision.DEFAULT)
        rn2 = _row_sum_sq(r)
        dist = (rn2 - 2.0 * mm) + cn2_ref[i][None, :]
        minv = jnp.min(dist, axis=1, keepdims=True)
        idxf = jnp.min(jnp.where(dist == minv, iota_f, jnp.float32(K)),
                       axis=1, keepdims=True)
        oh = (iota_f == idxf).astype(jnp.float32)
        ohb = oh.astype(jnp.bfloat16)
        q = ((jax.lax.dot_general(ohb, cbh_ref[i], dims,
                                  preferred_element_type=jnp.float32)
              + jax.lax.dot_general(ohb, cbm_ref[i], dims,
                                    preferred_element_type=jnp.float32))
             + jax.lax.dot_general(ohb, cbl_ref[i], dims,
                                   preferred_element_type=jnp.float32))
        d = q - r
        qo = qo + (r + d)
        commit_ref[i, :] += jnp.sum(d * d, axis=0)
        counts_ref[i, :] += jnp.sum(oh, axis=0)
        r = r - q
        emb_ref[0, i, :] = idxf[:, 0].astype(jnp.int32)
        layers_ref[i, 0, :, :] = qo.T
    zq_ref[0] = layers_ref[NQ - 1, 0, :, :]


@functools.partial(jax.jit, static_argnames=())
def kernel(z, codebooks):
    cn2 = jnp.stack([jnp.sum(codebooks[i] ** 2, axis=-1)
                     for i in range(NQ)])
    layers, zq, emb, counts, commit = pl.pallas_call(
        _rvq_kernel,
        grid=(B, NT),
        in_specs=[
            pl.BlockSpec((1, D, TT), lambda b, t: (b, 0, t)),
            pl.BlockSpec((NQ, K, D), lambda b, t: (0, 0, 0)),
            pl.BlockSpec((NQ, K), lambda b, t: (0, 0)),
        ],
        out_specs=[
            pl.BlockSpec((NQ, 1, D, TT), lambda b, t: (0, b, 0, t)),
            pl.BlockSpec((1, D, TT), lambda b, t: (b, 0, t)),
            pl.BlockSpec((1, NQ, TT), lambda b, t: (b, 0, t)),
            pl.BlockSpec((NQ, K), lambda b, t: (0, 0)),
            pl.BlockSpec((NQ, D), lambda b, t: (0, 0)),
        ],
        out_shape=[
            jax.ShapeDtypeStruct((NQ, B, D, T), jnp.float32),
            jax.ShapeDtypeStruct((B, D, T), jnp.float32),
            jax.ShapeDtypeStruct((B, NQ, T), jnp.int32),
            jax.ShapeDtypeStruct((NQ, K), jnp.float32),
            jax.ShapeDtypeStruct((NQ, D), jnp.float32),
        ],
        scratch_shapes=[pltpu.VMEM((NQ, K, D), jnp.bfloat16),
                        pltpu.VMEM((NQ, K, D), jnp.bfloat16),
                        pltpu.VMEM((NQ, K, D), jnp.bfloat16)],
        compiler_params=pltpu.CompilerParams(
            dimension_semantics=("arbitrary", "arbitrary")),
    )(z, codebooks, cn2)

    embed_nums = jnp.transpose(emb, (1, 0, 2))
    vqloss = jnp.sum(commit) / jnp.float32(B * T * D)
    probs = counts / jnp.float32(B * T)
    perps = jnp.exp(-jnp.sum(probs * jnp.log(probs + 1e-10), axis=1))
    perplexity = jnp.mean(perps)
    return (zq, embed_nums, vqloss, perplexity, layers)


# speedup vs baseline: 1.4966x; 1.1500x over previous
import functools

import jax
import jax.numpy as jnp
from jax.experimental import pallas as pl
from jax.experimental.pallas import tpu as pltpu

B, D, T = 16, 256, 1024
NQ, K = 8, 1024
TT = 512
NT = T // TT


def _row_sum_sq(v):
    vt = v.T
    rr = vt * vt
    pre = rr[0:128, :] + rr[128:256, :]
    acc = pre[0:8, :]
    for a in range(1, 16):
        acc = acc + pre[8 * a:8 * a + 8, :]
    s = acc[0:4, :] + acc[4:8, :]
    s = s[0:2, :] + s[2:4, :]
    return (s[0:1, :] + s[1:2, :]).T


def _rvq_kernel(z_ref, cb_ref, cn2_ref, layers_ref, zq_ref, emb_ref,
                counts_ref, commit_ref, cbh_ref, cbm_ref, cbl_ref):
    first = (pl.program_id(0) == 0) & (pl.program_id(1) == 0)

    @pl.when(first)
    def _init():
        counts_ref[...] = jnp.zeros_like(counts_ref)
        commit_ref[...] = jnp.zeros_like(commit_ref)
        cb = cb_ref[...]
        hi = cb.astype(jnp.bfloat16)
        rem = cb - hi.astype(jnp.float32)
        mid = rem.astype(jnp.bfloat16)
        lo = (rem - mid.astype(jnp.float32)).astype(jnp.bfloat16)
        cbh_ref[...] = hi
        cbm_ref[...] = mid
        cbl_ref[...] = lo

    x = z_ref[0]
    r = x.T
    qo = jnp.zeros((TT, D), jnp.float32)
    iota_f = jax.lax.broadcasted_iota(jnp.int32, (TT, K), 1).astype(jnp.float32)
    dims = (((1,), (0,)), ((), ()))
    for i in range(NQ):
        cb = cb_ref[i]
        mm = jax.lax.dot_general(
            r, cb, (((1,), (1,)), ((), ())),
            precision=jax.lax.Precision.DEFAULT)
        rn2 = _row_sum_sq(r)
        dist = (rn2 - 2.0 * mm) + cn2_ref[i][None, :]
        minv = jnp.min(dist, axis=1, keepdims=True)
        idxf = jnp.min(jnp.where(dist == minv, iota_f, jnp.float32(K)),
                       axis=1, keepdims=True)
        oh = (iota_f == idxf).astype(jnp.float32)
        ohb = oh.astype(jnp.bfloat16)
        q = ((jax.lax.dot_general(ohb, cbh_ref[i], dims,
                                  preferred_element_type=jnp.float32)
              + jax.lax.dot_general(ohb, cbm_ref[i], dims,
                                    preferred_element_type=jnp.float32))
             + jax.lax.dot_general(ohb, cbl_ref[i], dims,
                                   preferred_element_type=jnp.float32))
        d = q - r
        qo = qo + (r + d)
        commit_ref[i, :] += jnp.sum(d * d, axis=0)
        counts_ref[i, :] += jnp.sum(oh, axis=0)
        r = r - q
        emb_ref[0, i, :] = idxf[:, 0].astype(jnp.int32)
        layers_ref[i, 0, :, :] = qo.T
    zq_ref[0] = layers_ref[NQ - 1, 0, :, :]


@functools.partial(jax.jit, static_argnames=())
def kernel(z, codebooks):
    cn2 = jnp.stack([jnp.sum(codebooks[i] ** 2, axis=-1)
                     for i in range(NQ)])
    layers, zq, emb, counts, commit = pl.pallas_call(
        _rvq_kernel,
        grid=(B, NT),
        in_specs=[
            pl.BlockSpec((1, D, TT), lambda b, t: (b, 0, t)),
            pl.BlockSpec((NQ, K, D), lambda b, t: (0, 0, 0)),
            pl.BlockSpec((NQ, K), lambda b, t: (0, 0)),
        ],
        out_specs=[
            pl.BlockSpec((NQ, 1, D, TT), lambda b, t: (0, b, 0, t)),
            pl.BlockSpec((1, D, TT), lambda b, t: (b, 0, t)),
            pl.BlockSpec((1, NQ, TT), lambda b, t: (b, 0, t)),
            pl.BlockSpec((NQ, K), lambda b, t: (0, 0)),
            pl.BlockSpec((NQ, D), lambda b, t: (0, 0)),
        ],
        out_shape=[
            jax.ShapeDtypeStruct((NQ, B, D, T), jnp.float32),
            jax.ShapeDtypeStruct((B, D, T), jnp.float32),
            jax.ShapeDtypeStruct((B, NQ, T), jnp.int32),
            jax.ShapeDtypeStruct((NQ, K), jnp.float32),
            jax.ShapeDtypeStruct((NQ, D), jnp.float32),
        ],
        scratch_shapes=[pltpu.VMEM((NQ, K, D), jnp.bfloat16),
                        pltpu.VMEM((NQ, K, D), jnp.bfloat16),
                        pltpu.VMEM((NQ, K, D), jnp.bfloat16)],
        compiler_params=pltpu.CompilerParams(
            dimension_semantics=("arbitrary", "arbitrary")),
    )(z, codebooks, cn2)

    embed_nums = jnp.transpose(emb, (1, 0, 2))
    vqloss = jnp.sum(commit) / jnp.float32(B * T * D)
    probs = counts / jnp.float32(B * T)
    perps = jnp.exp(-jnp.sum(probs * jnp.log(probs + 1e-10), axis=1))
    perplexity = jnp.mean(perps)
    return (zq, embed_nums, vqloss, perplexity, layers)
